# Initial kernel scaffold; baseline (speedup 1.0000x reference)
import jax
import jax.numpy as jnp
from jax.experimental import pallas as pl


def kernel(feats, node):
    raise NotImplementedError("write your pallas kernel here")


# speedup vs baseline: 13.3827x; 13.3827x over previous
import functools

import jax
import jax.numpy as jnp
from jax.experimental import pallas as pl

_LAMBDA = 0.8
_K = 8


def _norm_rows(x):
    n = jnp.sqrt(jnp.sum(x * x, axis=-1, keepdims=True))
    return x / jnp.maximum(n, 1e-12)


def _gam_kernel(a_ref, full_ref, o_ref, *, n):
    a = _norm_rows(a_ref[0])
    fb = _norm_rows(full_ref[0])

    scores = jax.lax.dot_general(
        a, fb, (((1,), (1,)), ((), ())), preferred_element_type=jnp.float32)

    rb = scores.shape[0]
    iota = jax.lax.broadcasted_iota(jnp.int32, (rb, n), 1)
    neg_inf = jnp.float32(-jnp.inf)

    s = scores
    keep = jnp.zeros((rb, n), dtype=jnp.bool_)
    for _ in range(_K):
        m = jnp.max(s, axis=-1, keepdims=True)
        cand = jnp.where(s == m, iota, n)
        first = jnp.min(cand, axis=-1, keepdims=True)
        sel = iota == first
        keep = jnp.logical_or(keep, sel)
        s = jnp.where(sel, neg_inf, s)

    masked = jnp.where(keep, scores, neg_inf)
    m8 = jnp.max(masked, axis=-1, keepdims=True)
    p = jnp.exp(masked - m8)
    w = p / jnp.sum(p, axis=-1, keepdims=True)

    att = jax.lax.dot_general(
        w, fb, (((1,), (0,)), ((), ())), preferred_element_type=jnp.float32)
    o_ref[0] = _norm_rows(a * _LAMBDA + att * (1.0 - _LAMBDA))


@jax.jit
def kernel(feats, node):
    del node
    b, n, d = feats.shape
    rb = 256
    out = pl.pallas_call(
        functools.partial(_gam_kernel, n=n),
        grid=(b, n // rb),
        in_specs=[
            pl.BlockSpec((1, rb, d), lambda i, j: (i, j, 0)),
            pl.BlockSpec((1, n, d), lambda i, j: (i, 0, 0)),
        ],
        out_specs=pl.BlockSpec((1, rb, d), lambda i, j: (i, j, 0)),
        out_shape=jax.ShapeDtypeStruct((b, n, d), jnp.float32),
    )(feats, feats)
    return out


# speedup vs baseline: 28.3653x; 2.1195x over previous
import functools

import jax
import jax.numpy as jnp
from jax.experimental import pallas as pl

_LAMBDA = 0.8
_K = 8


def _norm_rows(x):
    n = jnp.sqrt(jnp.sum(x * x, axis=-1, keepdims=True))
    return x / jnp.maximum(n, 1e-12)


def _gam_kernel(a_ref, full_ref, o_ref, *, n):
    a = _norm_rows(a_ref[0])
    fb = _norm_rows(full_ref[0])

    scores = jax.lax.dot_general(
        a, fb, (((1,), (1,)), ((), ())), preferred_element_type=jnp.float32)

    neg_inf = jnp.float32(-jnp.inf)

    s = scores
    for _ in range(_K):
        m = jnp.max(s, axis=-1, keepdims=True)
        s = jnp.where(s == m, neg_inf, s)

    masked = jnp.where(s == neg_inf, scores, neg_inf)
    m8 = jnp.max(masked, axis=-1, keepdims=True)
    p = jnp.exp(masked - m8)
    w = p / jnp.sum(p, axis=-1, keepdims=True)

    att = jax.lax.dot_general(
        w, fb, (((1,), (0,)), ((), ())), preferred_element_type=jnp.float32)
    o_ref[0] = _norm_rows(a * _LAMBDA + att * (1.0 - _LAMBDA))


@jax.jit
def kernel(feats, node):
    del node
    b, n, d = feats.shape
    rb = 256
    out = pl.pallas_call(
        functools.partial(_gam_kernel, n=n),
        grid=(b, n // rb),
        in_specs=[
            pl.BlockSpec((1, rb, d), lambda i, j: (i, j, 0)),
            pl.BlockSpec((1, n, d), lambda i, j: (i, 0, 0)),
        ],
        out_specs=pl.BlockSpec((1, rb, d), lambda i, j: (i, j, 0)),
        out_shape=jax.ShapeDtypeStruct((b, n, d), jnp.float32),
    )(feats, feats)
    return out


# speedup vs baseline: 28.6004x; 1.0083x over previous
import functools

import jax
import jax.numpy as jnp
from jax.experimental import pallas as pl

_LAMBDA = 0.8
_K = 8


def _norm_rows(x):
    n = jnp.sqrt(jnp.sum(x * x, axis=-1, keepdims=True))
    return x / jnp.maximum(n, 1e-12)


def _normalize_kernel(x_ref, o_ref):
    o_ref[0] = _norm_rows(x_ref[0])


def _gam_kernel(a_ref, full_ref, o_ref):
    a = a_ref[0]
    fb = full_ref[0]

    scores = jax.lax.dot_general(
        a, fb, (((1,), (1,)), ((), ())), preferred_element_type=jnp.float32)

    neg_inf = jnp.float32(-jnp.inf)

    m1 = jnp.max(scores, axis=-1, keepdims=True)
    s = jnp.where(scores == m1, neg_inf, scores)
    for _ in range(_K - 1):
        m = jnp.max(s, axis=-1, keepdims=True)
        s = jnp.where(s == m, neg_inf, s)

    p = jnp.where(s == neg_inf, jnp.exp(scores - m1), 0.0)
    w = p * (1.0 / jnp.sum(p, axis=-1, keepdims=True))

    att = jax.lax.dot_general(
        w, fb, (((1,), (0,)), ((), ())), preferred_element_type=jnp.float32)
    o_ref[0] = _norm_rows(a * _LAMBDA + att * (1.0 - _LAMBDA))


@jax.jit
def kernel(feats, node):
    del node
    b, n, d = feats.shape
    f = pl.pallas_call(
        _normalize_kernel,
        grid=(b,),
        in_specs=[pl.BlockSpec((1, n, d), lambda i: (i, 0, 0))],
        out_specs=pl.BlockSpec((1, n, d), lambda i: (i, 0, 0)),
        out_shape=jax.ShapeDtypeStruct((b, n, d), jnp.float32),
    )(feats)
    rb = 256
    out = pl.pallas_call(
        _gam_kernel,
        grid=(b, n // rb),
        in_specs=[
            pl.BlockSpec((1, rb, d), lambda i, j: (i, j, 0)),
            pl.BlockSpec((1, n, d), lambda i, j: (i, 0, 0)),
        ],
        out_specs=pl.BlockSpec((1, rb, d), lambda i, j: (i, j, 0)),
        out_shape=jax.ShapeDtypeStruct((b, n, d), jnp.float32),
    )(f, f)
    return out


# speedup vs baseline: 29.6053x; 1.0351x over previous
import functools

import jax
import jax.numpy as jnp
from jax.experimental import pallas as pl

_LAMBDA = 0.8
_K = 8


def _norm_rows(x):
    n = jnp.sqrt(jnp.sum(x * x, axis=-1, keepdims=True))
    return x / jnp.maximum(n, 1e-12)


def _normalize_kernel(x_ref, o_ref):
    o_ref[0] = _norm_rows(x_ref[0])


def _gam_kernel(a_ref, full_ref, o_ref):
    a = a_ref[0]
    fb = full_ref[0]

    scores = jax.lax.dot_general(
        a, fb, (((1,), (1,)), ((), ())), preferred_element_type=jnp.float32)

    neg_inf = jnp.float32(-jnp.inf)

    m1 = jnp.max(scores, axis=-1, keepdims=True)
    m = m1
    for _ in range(_K - 1):
        m = jnp.max(jnp.where(scores >= m, neg_inf, scores),
                    axis=-1, keepdims=True)

    p = jnp.where(scores >= m, jnp.exp(scores - m1), 0.0)
    w = p * (1.0 / jnp.sum(p, axis=-1, keepdims=True))

    att = jax.lax.dot_general(
        w, fb, (((1,), (0,)), ((), ())), preferred_element_type=jnp.float32)
    o_ref[0] = _norm_rows(a * _LAMBDA + att * (1.0 - _LAMBDA))


@jax.jit
def kernel(feats, node):
    del node
    b, n, d = feats.shape
    f = pl.pallas_call(
        _normalize_kernel,
        grid=(b,),
        in_specs=[pl.BlockSpec((1, n, d), lambda i: (i, 0, 0))],
        out_specs=pl.BlockSpec((1, n, d), lambda i: (i, 0, 0)),
        out_shape=jax.ShapeDtypeStruct((b, n, d), jnp.float32),
    )(feats)
    rb = 256
    out = pl.pallas_call(
        _gam_kernel,
        grid=(b, n // rb),
        in_specs=[
            pl.BlockSpec((1, rb, d), lambda i, j: (i, j, 0)),
            pl.BlockSpec((1, n, d), lambda i, j: (i, 0, 0)),
        ],
        out_specs=pl.BlockSpec((1, rb, d), lambda i, j: (i, j, 0)),
        out_shape=jax.ShapeDtypeStruct((b, n, d), jnp.float32),
    )(f, f)
    return out


# speedup vs baseline: 31.2305x; 1.0549x over previous
import functools

import jax
import jax.numpy as jnp
from jax.experimental import pallas as pl

_LAMBDA = 0.8
_K = 8


def _norm_rows(x):
    n = jnp.sqrt(jnp.sum(x * x, axis=-1, keepdims=True))
    return x / jnp.maximum(n, 1e-12)


def _normalize_kernel(x_ref, o_ref):
    o_ref[0] = _norm_rows(x_ref[0])


def _gam_kernel(a_ref, full_ref, o_ref):
    a = a_ref[0]
    fb = full_ref[0]

    scores = jax.lax.dot_general(
        a, fb, (((1,), (1,)), ((), ())), preferred_element_type=jnp.float32)

    neg_inf = jnp.float32(-jnp.inf)

    m = jnp.max(scores, axis=-1, keepdims=True)
    for _ in range(_K - 1):
        m = jnp.max(jnp.where(scores >= m, neg_inf, scores),
                    axis=-1, keepdims=True)

    p = jnp.where(scores >= m, jnp.exp(scores), 0.0)
    att = jax.lax.dot_general(
        p, fb, (((1,), (0,)), ((), ())), preferred_element_type=jnp.float32)
    att = att * (1.0 / jnp.sum(p, axis=-1, keepdims=True))
    o_ref[0] = _norm_rows(a * _LAMBDA + att * (1.0 - _LAMBDA))


@jax.jit
def kernel(feats, node):
    del node
    b, n, d = feats.shape
    f = pl.pallas_call(
        _normalize_kernel,
        grid=(b,),
        in_specs=[pl.BlockSpec((1, n, d), lambda i: (i, 0, 0))],
        out_specs=pl.BlockSpec((1, n, d), lambda i: (i, 0, 0)),
        out_shape=jax.ShapeDtypeStruct((b, n, d), jnp.float32),
    )(feats)
    rb = 256
    out = pl.pallas_call(
        _gam_kernel,
        grid=(b, n // rb),
        in_specs=[
            pl.BlockSpec((1, rb, d), lambda i, j: (i, j, 0)),
            pl.BlockSpec((1, n, d), lambda i, j: (i, 0, 0)),
        ],
        out_specs=pl.BlockSpec((1, rb, d), lambda i, j: (i, j, 0)),
        out_shape=jax.ShapeDtypeStruct((b, n, d), jnp.float32),
    )(f, f)
    return out


# speedup vs baseline: 33.1989x; 1.0630x over previous
import functools

import jax
import jax.numpy as jnp
from jax.experimental import pallas as pl

_LAMBDA = 0.8
_K = 8


def _norm_rows(x):
    n = jnp.sqrt(jnp.sum(x * x, axis=-1, keepdims=True))
    return x / jnp.maximum(n, 1e-12)


def _normalize_kernel(x_ref, o_ref):
    o_ref[0] = _norm_rows(x_ref[0])


def _gam_kernel(a_ref, full_ref, o_ref):
    a = a_ref[0]
    fb = full_ref[0]

    scores = jax.lax.dot_general(
        a, fb, (((1,), (1,)), ((), ())), preferred_element_type=jnp.float32)

    neg_inf = jnp.float32(-jnp.inf)

    m = jnp.max(scores, axis=-1, keepdims=True)
    for _ in range(_K - 1):
        m = jnp.max(jnp.where(scores >= m, neg_inf, scores),
                    axis=-1, keepdims=True)

    p = jnp.where(scores >= m, jnp.exp(scores), 0.0)
    att = jax.lax.dot_general(
        p, fb, (((1,), (0,)), ((), ())), preferred_element_type=jnp.float32)
    att = att * (1.0 / jnp.sum(p, axis=-1, keepdims=True))
    o_ref[0] = _norm_rows(a * _LAMBDA + att * (1.0 - _LAMBDA))


@jax.jit
def kernel(feats, node):
    del node
    b, n, d = feats.shape
    f = pl.pallas_call(
        _normalize_kernel,
        grid=(b,),
        in_specs=[pl.BlockSpec((1, n, d), lambda i: (i, 0, 0))],
        out_specs=pl.BlockSpec((1, n, d), lambda i: (i, 0, 0)),
        out_shape=jax.ShapeDtypeStruct((b, n, d), jnp.float32),
    )(feats)
    rb = 512
    out = pl.pallas_call(
        _gam_kernel,
        grid=(b, n // rb),
        in_specs=[
            pl.BlockSpec((1, rb, d), lambda i, j: (i, j, 0)),
            pl.BlockSpec((1, n, d), lambda i, j: (i, 0, 0)),
        ],
        out_specs=pl.BlockSpec((1, rb, d), lambda i, j: (i, j, 0)),
        out_shape=jax.ShapeDtypeStruct((b, n, d), jnp.float32),
    )(f, f)
    return out


# speedup vs baseline: 34.8700x; 1.0503x over previous
import functools

import jax
import jax.numpy as jnp
from jax.experimental import pallas as pl

_LAMBDA = 0.8
_K = 8


def _norm_rows(x):
    n = jnp.sqrt(jnp.sum(x * x, axis=-1, keepdims=True))
    return x / jnp.maximum(n, 1e-12)


def _normalize_kernel(x_ref, o_ref):
    o_ref[0] = _norm_rows(x_ref[0])


def _gam_kernel(a_ref, full_ref, o_ref):
    a = a_ref[0]
    fb = full_ref[0]

    scores = jax.lax.dot_general(
        a, fb, (((1,), (1,)), ((), ())), preferred_element_type=jnp.float32)

    neg_inf = jnp.float32(-jnp.inf)

    m = jnp.max(scores, axis=-1, keepdims=True)
    for _ in range(_K - 1):
        m = jnp.max(jnp.where(scores >= m, neg_inf, scores),
                    axis=-1, keepdims=True)

    p = jnp.where(scores >= m, jnp.exp(scores), 0.0)
    att = jax.lax.dot_general(
        p, fb, (((1,), (0,)), ((), ())), preferred_element_type=jnp.float32)
    att = att * (1.0 / jnp.sum(p, axis=-1, keepdims=True))
    o_ref[0] = _norm_rows(a * _LAMBDA + att * (1.0 - _LAMBDA))


@jax.jit
def kernel(feats, node):
    del node
    b, n, d = feats.shape
    f = pl.pallas_call(
        _normalize_kernel,
        grid=(b,),
        in_specs=[pl.BlockSpec((1, n, d), lambda i: (i, 0, 0))],
        out_specs=pl.BlockSpec((1, n, d), lambda i: (i, 0, 0)),
        out_shape=jax.ShapeDtypeStruct((b, n, d), jnp.float32),
    )(feats)
    rb = 1024
    out = pl.pallas_call(
        _gam_kernel,
        grid=(b, n // rb),
        in_specs=[
            pl.BlockSpec((1, rb, d), lambda i, j: (i, j, 0)),
            pl.BlockSpec((1, n, d), lambda i, j: (i, 0, 0)),
        ],
        out_specs=pl.BlockSpec((1, rb, d), lambda i, j: (i, j, 0)),
        out_shape=jax.ShapeDtypeStruct((b, n, d), jnp.float32),
    )(f, f)
    return out


# speedup vs baseline: 34.8719x; 1.0001x over previous
import functools

import jax
import jax.numpy as jnp
from jax.experimental import pallas as pl

_LAMBDA = 0.8
_K = 8


def _norm_rows(x):
    n = jnp.sqrt(jnp.sum(x * x, axis=-1, keepdims=True))
    return x / jnp.maximum(n, 1e-12)


def _normalize_kernel(x_ref, o_ref):
    o_ref[0] = _norm_rows(x_ref[0])


def _gam_kernel(a_ref, full_ref, o_ref):
    a = a_ref[0]
    fb = full_ref[0]

    scores = jax.lax.dot_general(
        a, fb, (((1,), (1,)), ((), ())), preferred_element_type=jnp.float32)

    neg_inf = jnp.float32(-jnp.inf)

    m = jnp.max(scores, axis=-1, keepdims=True)
    for _ in range(_K - 1):
        m = jnp.max(jnp.where(scores >= m, neg_inf, scores),
                    axis=-1, keepdims=True)

    p = jnp.where(scores >= m, jnp.exp(scores), 0.0)
    att = jax.lax.dot_general(
        p, fb, (((1,), (0,)), ((), ())), preferred_element_type=jnp.float32)
    att = att * (1.0 / jnp.sum(p, axis=-1, keepdims=True))
    o_ref[0] = _norm_rows(a * _LAMBDA + att * (1.0 - _LAMBDA))


@jax.jit
def kernel(feats, node):
    del node
    b, n, d = feats.shape
    f = pl.pallas_call(
        _normalize_kernel,
        grid=(b,),
        in_specs=[pl.BlockSpec((1, n, d), lambda i: (i, 0, 0))],
        out_specs=pl.BlockSpec((1, n, d), lambda i: (i, 0, 0)),
        out_shape=jax.ShapeDtypeStruct((b, n, d), jnp.float32),
    )(feats)
    rb = 2048
    out = pl.pallas_call(
        _gam_kernel,
        grid=(b, n // rb),
        in_specs=[
            pl.BlockSpec((1, rb, d), lambda i, j: (i, j, 0)),
            pl.BlockSpec((1, n, d), lambda i, j: (i, 0, 0)),
        ],
        out_specs=pl.BlockSpec((1, rb, d), lambda i, j: (i, j, 0)),
        out_shape=jax.ShapeDtypeStruct((b, n, d), jnp.float32),
    )(f, f)
    return out
